# Initial kernel scaffold; baseline (speedup 1.0000x reference)
import jax
import jax.numpy as jnp
from jax.experimental import pallas as pl


def kernel(x, edge_index, batch_graph, W_out, b_out, W_root, W_fc, b_fc, W1, b1, W2, b2, Wo, bo):
    raise NotImplementedError("write your pallas kernel here")


# speedup vs baseline: 16.0492x; 16.0492x over previous
import functools

import jax
import jax.numpy as jnp
from jax import lax
from jax.experimental import pallas as pl
from jax.experimental.pallas import tpu as pltpu
from jax.experimental.pallas import tpu_sc as plsc

N = 10000
C = 128
E = 320000
NPAD = 10016
NC = 2
NS = 16
NW = NC * NS
CHUNK = 128
NCHUNKS = E // CHUNK
BASE_CH = NCHUNKS // NW
EXTRA_CH = NCHUNKS % NW
GROUPS = NPAD // 16
BASE_GR = GROUPS // NS
EXTRA_GR = GROUPS % NS


@functools.cache
def _sc_gather_scatter_kernel():
    return functools.partial(
        pl.kernel,
        mesh=plsc.VectorSubcoreMesh(core_axis_name="c", subcore_axis_name="s"),
        out_type=[
            jax.ShapeDtypeStruct((NC, NPAD, C), jnp.float32),
            jax.ShapeDtypeStruct((NC, NPAD), jnp.float32),
        ],
        scratch_types=[
            pltpu.VMEM_SHARED((NPAD, C), jnp.float32),
            pltpu.VMEM_SHARED((NPAD,), jnp.float32),
            pltpu.VMEM((CHUNK,), jnp.int32),
            pltpu.VMEM((CHUNK,), jnp.int32),
            pltpu.VMEM((CHUNK, C), jnp.float32),
            pltpu.VMEM((16, C), jnp.float32),
            pltpu.VMEM((2048,), jnp.float32),
            pltpu.VMEM((CHUNK,), jnp.float32),
            pltpu.SemaphoreType.DMA,
        ],
    )(_sc_body)


def _sc_body(x_hbm, row_hbm, col_hbm, p_hbm, deg_hbm,
             acc, degs, row_v, col_v, rows_v, zrow, zflat, ones_v,
             sem):
    cid = lax.axis_index("c")
    sid = lax.axis_index("s")
    wid = sid * NC + cid

    z16 = jnp.zeros((16,), jnp.float32)
    for i in range(16):
        for j in range(C // 16):
            zrow[i, pl.ds(j * 16, 16)] = z16
    for j in range(CHUNK // 16):
        ones_v[pl.ds(j * 16, 16)] = jnp.ones((16,), jnp.float32)

    def _zflat_body(i, carry):
        zflat[pl.ds(pl.multiple_of(i * 16, 16), 16)] = z16
        return carry
    lax.fori_loop(0, 2048 // 16, _zflat_body, 0)

    ngr = BASE_GR + jnp.where(sid < EXTRA_GR, 1, 0)

    def _zero_body(k, carry):
        g = sid + k * NS
        off = pl.multiple_of(g * 16, 16)
        pltpu.sync_copy(zrow, acc.at[pl.ds(off, 16)])
        return carry
    lax.fori_loop(0, ngr, _zero_body, 0)

    @pl.when(sid == 0)
    def _():
        for off, size in ((0, 2048), (2048, 2048), (4096, 2048),
                          (6144, 2048), (8192, NPAD - 8192)):
            pltpu.sync_copy(zflat.at[pl.ds(0, size)], degs.at[pl.ds(off, size)])

    plsc.subcore_barrier()

    nch = BASE_CH + jnp.where(wid < EXTRA_CH, 1, 0)

    def _chunk_body(k, carry):
        ci = wid + k * NW
        off = pl.multiple_of(ci * CHUNK, CHUNK)
        pltpu.sync_copy(row_hbm.at[pl.ds(off, CHUNK)], row_v)
        pltpu.sync_copy(col_hbm.at[pl.ds(off, CHUNK)], col_v)
        for j in range(CHUNK // 16):
            r = row_v[pl.ds(j * 16, 16)]
            c = col_v[pl.ds(j * 16, 16)]
            col_v[pl.ds(j * 16, 16)] = jnp.where(r == c, N, c)
        pltpu.async_copy(x_hbm.at[row_v], rows_v, sem).wait()
        pltpu.sync_copy(rows_v, acc.at[col_v], add=True)
        pltpu.sync_copy(ones_v, degs.at[col_v], add=True)
        return carry
    lax.fori_loop(0, nch, _chunk_body, 0)

    plsc.subcore_barrier()

    def _wb_body(k, carry):
        g = sid + k * NS
        off = pl.multiple_of(g * 16, 16)
        pltpu.sync_copy(acc.at[pl.ds(off, 16)], p_hbm.at[cid, pl.ds(off, 16)])
        return carry
    lax.fori_loop(0, ngr, _wb_body, 0)

    @pl.when(sid == 0)
    def _():
        pltpu.sync_copy(degs, deg_hbm.at[cid])


BN = 400


def _tc_body(x_ref, p_ref, d_ref, woutT, bout, wrootT, wfcT, bfc,
             w1T, b1, w2T, b2, woT, bo, o_ref):
    xb = x_ref[...]
    psum = p_ref[0] + p_ref[1]
    d = d_ref[...]
    deg = 1.0 + d[:, 0:1] + d[:, 1:2]
    agg = (xb + psum) / deg
    h = (jnp.dot(agg, woutT[...], preferred_element_type=jnp.float32)
         + jnp.dot(xb, wrootT[...], preferred_element_type=jnp.float32)
         + bout[...])
    h = jnp.maximum(h, 0.0)
    h = jnp.dot(h, wfcT[...], preferred_element_type=jnp.float32) + bfc[...]
    h = jnp.where(h >= 0, h, 0.01 * h)
    h = jnp.dot(h, w1T[...], preferred_element_type=jnp.float32) + b1[...]
    h = jnp.dot(h, w2T[...], preferred_element_type=jnp.float32) + b2[...]
    h = jnp.dot(h, woT[...], preferred_element_type=jnp.float32) + bo[...]
    o_ref[...] = jax.nn.sigmoid(h)


def _tc_head(x, p, dT, woutT, bout, wrootT, wfcT, bfc, w1T, b1, w2T, b2,
             woT, bo, *, interpret=False):
    grid = N // BN
    full = lambda i: (0, 0)
    return pl.pallas_call(
        _tc_body,
        grid=(grid,),
        in_specs=[
            pl.BlockSpec((BN, C), lambda i: (i, 0)),
            pl.BlockSpec((NC, BN, C), lambda i: (0, i, 0)),
            pl.BlockSpec((BN, NC), lambda i: (i, 0)),
            pl.BlockSpec((C, C), full),
            pl.BlockSpec((1, C), full),
            pl.BlockSpec((C, C), full),
            pl.BlockSpec((C, C), full),
            pl.BlockSpec((1, C), full),
            pl.BlockSpec((C, C), full),
            pl.BlockSpec((1, C), full),
            pl.BlockSpec((C, 64), full),
            pl.BlockSpec((1, 64), full),
            pl.BlockSpec((64, 6), full),
            pl.BlockSpec((1, 6), full),
        ],
        out_specs=pl.BlockSpec((BN, 6), lambda i: (i, 0)),
        out_shape=jax.ShapeDtypeStruct((N, 6), jnp.float32),
        interpret=interpret,
    )(x, p, dT, woutT, bout, wrootT, wfcT, bfc, w1T, b1, w2T, b2, woT, bo)


def kernel(x, edge_index, batch_graph, W_out, b_out, W_root, W_fc, b_fc,
           W1, b1, W2, b2, Wo, bo):
    row = edge_index[0]
    col = edge_index[1]
    p, dpart = _sc_gather_scatter_kernel()(x, row, col)
    return _tc_head(
        x, p, dpart.T,
        W_out.T, b_out.reshape(1, -1),
        W_root.T,
        W_fc.T, b_fc.reshape(1, -1),
        W1.T, b1.reshape(1, -1),
        W2.T, b2.reshape(1, -1),
        Wo.T, bo.reshape(1, -1),
    )
